# baseline (speedup 1.0000x reference)
import jax
import jax.numpy as jnp
from jax.experimental import pallas as pl
from jax.experimental.pallas import tpu as pltpu

EMBED = 768
HIDDEN = 3072
E = 8
TOPK = 2
BT = 512


def _moe_body(x_ref, wg_ref, w1_ref, b1_ref, w2_ref, b2_ref, out_ref, gates_ref):
    e = pl.program_id(1)

    @pl.when(e == 0)
    def _():
        logits = jax.lax.dot_general(
            x_ref[...], wg_ref[...], (((1,), (1,)), ((), ())),
            preferred_element_type=jnp.float32)
        m = jnp.max(logits, axis=-1, keepdims=True)
        p = jnp.exp(logits - m)
        probs = p / jnp.sum(p, axis=-1, keepdims=True)
        cols = jax.lax.broadcasted_iota(jnp.int32, probs.shape, 1)
        i1 = jnp.argmax(probs, axis=-1)[:, None]
        m1 = cols == i1
        p1 = jnp.max(probs, axis=-1, keepdims=True)
        probs2 = jnp.where(m1, -jnp.inf, probs)
        i2 = jnp.argmax(probs2, axis=-1)[:, None]
        m2 = cols == i2
        p2 = jnp.max(probs2, axis=-1, keepdims=True)
        denom = p1 + p2 + 1e-9
        gates_ref[...] = (jnp.where(m1, p1 / denom, 0.0)
                          + jnp.where(m2, p2 / denom, 0.0))

    h = jax.lax.dot_general(
        x_ref[...], w1_ref[0], (((1,), (1,)), ((), ())),
        preferred_element_type=jnp.float32)
    h = jnp.maximum(h + b1_ref[0], 0.0)
    y = jax.lax.dot_general(
        h, w2_ref[0], (((1,), (1,)), ((), ())),
        preferred_element_type=jnp.float32)
    y = y + b2_ref[0]

    ecol = jax.lax.broadcasted_iota(jnp.int32, (BT, E), 1) == e
    g = jnp.sum(jnp.where(ecol, gates_ref[...], 0.0), axis=-1, keepdims=True)

    @pl.when(e == 0)
    def _():
        out_ref[...] = g * y

    @pl.when(e != 0)
    def _():
        out_ref[...] += g * y


def kernel(x, Wg, W1, b1, W2, b2):
    orig_shape = x.shape
    xf = x.reshape(-1, EMBED)
    T = xf.shape[0]
    out = pl.pallas_call(
        _moe_body,
        grid=(T // BT, E),
        in_specs=[
            pl.BlockSpec((BT, EMBED), lambda i, e: (i, 0)),
            pl.BlockSpec((E, EMBED), lambda i, e: (0, 0)),
            pl.BlockSpec((1, HIDDEN, EMBED), lambda i, e: (e, 0, 0)),
            pl.BlockSpec((1, 1, HIDDEN), lambda i, e: (e, 0, 0)),
            pl.BlockSpec((1, EMBED, HIDDEN), lambda i, e: (e, 0, 0)),
            pl.BlockSpec((1, 1, EMBED), lambda i, e: (e, 0, 0)),
        ],
        out_specs=pl.BlockSpec((BT, EMBED), lambda i, e: (i, 0)),
        out_shape=jax.ShapeDtypeStruct((T, EMBED), jnp.float32),
        scratch_shapes=[pltpu.VMEM((BT, E), jnp.float32)],
    )(xf, Wg, W1, b1.reshape(E, 1, HIDDEN), W2, b2.reshape(E, 1, EMBED))
    return out.reshape(orig_shape)
